# baseline (speedup 1.0000x reference)
import functools

import jax
import jax.numpy as jnp
from jax import lax
from jax.experimental import pallas as pl
from jax.experimental.pallas import tpu as pltpu
from jax.experimental.pallas import tpu_sc as plsc

M = 1024
N = 100000
D = 32
B = 16384

_NC = 2
_NS = 16
_NW = _NC * _NS
_BPW = B // _NW
_CH = 128
_NCH = _BPW // _CH
_ROWS_PER_W = B // _CH // _NW

_KB = 2048
_NKB = -(-N // _KB)


def _sc_body(i_hbm, j_hbm, wf_hbm, b_hbm, zeros_hbm,
             a2_out, bj_out,
             iidx, jidx, bjv, gidx, sidx, gbuf, gsem, bounce, shared_at):
    c = lax.axis_index("c")
    s = lax.axis_index("s")
    wid = s * _NC + c
    base = wid * _BPW

    for k in range(_NCH):
        pltpu.sync_copy(i_hbm.at[pl.ds(base + k * _CH, _CH)], iidx[k])
        pltpu.sync_copy(j_hbm.at[pl.ds(base + k * _CH, _CH)], jidx[k])

    @pl.when(s == 0)
    def _zero():
        pltpu.sync_copy(zeros_hbm, shared_at)

    plsc.subcore_barrier()

    for k in range(_NCH):
        pltpu.sync_copy(b_hbm.at[jidx[k]], bjv[k])
        pltpu.sync_copy(bjv[k], bj_out.at[pl.ds(base + k * _CH, _CH)])

    for k in range(_NCH):
        for d in range(D):
            for t in range(_CH // 16):
                sl = pl.ds(t * 16, 16)
                gidx[d, sl] = jidx[k][sl] * D + d
                sidx[d, sl] = iidx[k][sl] + d * M
        handles = [
            pltpu.async_copy(wf_hbm.at[gidx.at[d]], gbuf.at[d], gsem)
            for d in range(D)
        ]
        for h in handles:
            h.wait()
        for d in range(D):
            pltpu.sync_copy(gbuf.at[d], shared_at.at[sidx.at[d]], add=True)

    plsc.subcore_barrier()

    @pl.when(s == 0)
    def _publish():
        pltpu.sync_copy(shared_at, bounce)
        pltpu.sync_copy(bounce, a2_out.at[c])


@functools.cache
def _sc_call():
    return pl.kernel(
        _sc_body,
        out_type=[
            jax.ShapeDtypeStruct((_NC, D * M), jnp.float32),
            jax.ShapeDtypeStruct((B,), jnp.float32),
        ],
        mesh=plsc.VectorSubcoreMesh(
            core_axis_name="c", subcore_axis_name="s", num_cores=_NC),
        scratch_types=[
            [pltpu.VMEM((_CH,), jnp.int32) for _ in range(_NCH)],
            [pltpu.VMEM((_CH,), jnp.int32) for _ in range(_NCH)],
            [pltpu.VMEM((_CH,), jnp.float32) for _ in range(_NCH)],
            pltpu.VMEM((D, _CH), jnp.int32),
            pltpu.VMEM((D, _CH), jnp.int32),
            pltpu.VMEM((D, _CH), jnp.float32),
            pltpu.SemaphoreType.DMA,
            pltpu.VMEM((D * M,), jnp.float32),
            pltpu.VMEM_SHARED((D * M,), jnp.float32),
        ],
        compiler_params=pltpu.CompilerParams(use_tc_tiling_on_sc=False),
    )


def _mm_body(vt_ref, rt_ref, mut_ref, ht_ref, acc_ref):
    k = pl.program_id(0)

    @pl.when(k == 0)
    def _init():
        acc_ref[...] = jnp.zeros_like(acc_ref)

    base = k * _KB
    rows = lax.broadcasted_iota(jnp.int32, (_KB, M), 0) + base
    rt = jnp.where(rows < N, rt_ref[...], 0.0)
    cols = lax.broadcasted_iota(jnp.int32, (D, _KB), 1) + base
    vt = jnp.where(cols < N, vt_ref[...], 0.0)
    acc_ref[...] += jnp.dot(vt, rt, preferred_element_type=jnp.float32)

    @pl.when(k == pl.num_programs(0) - 1)
    def _epilogue():
        ht_ref[...] = jax.nn.sigmoid(acc_ref[...] + mut_ref[...])


def _mm_call(vt, rt, mut):
    return pl.pallas_call(
        _mm_body,
        grid=(_NKB,),
        in_specs=[
            pl.BlockSpec((D, _KB), lambda k: (0, k)),
            pl.BlockSpec((_KB, M), lambda k: (k, 0)),
            pl.BlockSpec((D, 1), lambda k: (0, 0)),
        ],
        out_specs=pl.BlockSpec((D, M), lambda k: (0, 0)),
        out_shape=jax.ShapeDtypeStruct((D, M), jnp.float32),
        scratch_shapes=[pltpu.VMEM((D, M), jnp.float32)],
        compiler_params=pltpu.CompilerParams(
            dimension_semantics=("arbitrary",),
        ),
    )(vt, rt, mut)


def _combine_body(ht_ref, a2t_ref, bj_ref, out_ref):
    at = a2t_ref[0] + a2t_ref[1]
    s = jnp.sum(ht_ref[...] * at)
    out_ref[...] = s + bj_ref[...]


def _combine_call(ht, a2t, bj):
    return pl.pallas_call(
        _combine_body,
        in_specs=[
            pl.BlockSpec((D, M), lambda: (0, 0)),
            pl.BlockSpec((_NC, D, M), lambda: (0, 0, 0)),
            pl.BlockSpec((B,), lambda: (0,)),
        ],
        out_specs=pl.BlockSpec((B,), lambda: (0,)),
        out_shape=jax.ShapeDtypeStruct((B,), jnp.float32),
    )(ht, a2t, bj)


def kernel(r, i, j, v, mu, w, b):
    i1 = i.astype(jnp.int32)
    j1 = j.astype(jnp.int32)
    wf = w.reshape(N * D)
    zeros = jnp.zeros((D * M,), jnp.float32)
    a2, bj = _sc_call()(i1, j1, wf, b, zeros)
    ht = _mm_call(v.T, r.T, mu.T)
    return _combine_call(ht, a2.reshape(_NC, D, M), bj)


# speedup vs baseline: 1.2207x; 1.2207x over previous
import functools

import jax
import jax.numpy as jnp
from jax import lax
from jax.experimental import pallas as pl
from jax.experimental.pallas import tpu as pltpu
from jax.experimental.pallas import tpu_sc as plsc

M = 1024
N = 100000
D = 32
B = 16384

_NC = 2
_NS = 16
_NW = _NC * _NS
_BPW = B // _NW
_CH = 128
_NCH = _BPW // _CH
_ROWS_PER_W = B // _CH // _NW

_KB = 2048
_NKB = -(-N // _KB)


def _sc_body(i_hbm, j_hbm, wf_hbm, b_hbm, zeros_hbm,
             a2_out, bj_out,
             iidx, jidx, bjv, gidx, sidx, gbuf, gsem, bounce, shared_at):
    c = lax.axis_index("c")
    s = lax.axis_index("s")
    wid = s * _NC + c
    base = wid * _BPW

    for k in range(_NCH):
        pltpu.sync_copy(i_hbm.at[pl.ds(base + k * _CH, _CH)], iidx[k])
        pltpu.sync_copy(j_hbm.at[pl.ds(base + k * _CH, _CH)], jidx[k])

    @pl.when(s == 0)
    def _zero():
        pltpu.sync_copy(zeros_hbm, shared_at)

    plsc.subcore_barrier()

    for k in range(_NCH):
        pltpu.sync_copy(b_hbm.at[jidx[k]], bjv[k])
        pltpu.sync_copy(bjv[k], bj_out.at[pl.ds(base + k * _CH, _CH)])

    for k in range(_NCH):
        for d in range(D):
            for t in range(_CH // 16):
                sl = pl.ds(t * 16, 16)
                gidx[d, sl] = jidx[k][sl] + d * N
                sidx[d, sl] = iidx[k][sl] + d * M
        handles = [
            pltpu.async_copy(wf_hbm.at[gidx.at[d]], gbuf.at[d], gsem)
            for d in range(D)
        ]
        for h in handles:
            h.wait()
        for d in range(D):
            pltpu.sync_copy(gbuf.at[d], shared_at.at[sidx.at[d]], add=True)

    plsc.subcore_barrier()

    @pl.when(s == 0)
    def _publish():
        pltpu.sync_copy(shared_at, bounce)
        pltpu.sync_copy(bounce, a2_out.at[c])


@functools.cache
def _sc_call():
    return pl.kernel(
        _sc_body,
        out_type=[
            jax.ShapeDtypeStruct((_NC, D * M), jnp.float32),
            jax.ShapeDtypeStruct((B,), jnp.float32),
        ],
        mesh=plsc.VectorSubcoreMesh(
            core_axis_name="c", subcore_axis_name="s", num_cores=_NC),
        scratch_types=[
            [pltpu.VMEM((_CH,), jnp.int32) for _ in range(_NCH)],
            [pltpu.VMEM((_CH,), jnp.int32) for _ in range(_NCH)],
            [pltpu.VMEM((_CH,), jnp.float32) for _ in range(_NCH)],
            pltpu.VMEM((D, _CH), jnp.int32),
            pltpu.VMEM((D, _CH), jnp.int32),
            pltpu.VMEM((D, _CH), jnp.float32),
            pltpu.SemaphoreType.DMA,
            pltpu.VMEM((D * M,), jnp.float32),
            pltpu.VMEM_SHARED((D * M,), jnp.float32),
        ],
        compiler_params=pltpu.CompilerParams(use_tc_tiling_on_sc=False),
    )


def _mm_body(vt_ref, rt_ref, mut_ref, ht_ref, acc_ref):
    k = pl.program_id(0)

    @pl.when(k == 0)
    def _init():
        acc_ref[...] = jnp.zeros_like(acc_ref)

    @pl.when(k < pl.num_programs(0) - 1)
    def _full_step():
        acc_ref[...] += jnp.dot(vt_ref[...], rt_ref[...],
                                preferred_element_type=jnp.float32)

    @pl.when(k == pl.num_programs(0) - 1)
    def _tail_step():
        base = k * _KB
        rows = lax.broadcasted_iota(jnp.int32, (_KB, M), 0) + base
        rt = jnp.where(rows < N, rt_ref[...], 0.0)
        cols = lax.broadcasted_iota(jnp.int32, (D, _KB), 1) + base
        vt = jnp.where(cols < N, vt_ref[...], 0.0)
        acc_ref[...] += jnp.dot(vt, rt, preferred_element_type=jnp.float32)

    @pl.when(k == pl.num_programs(0) - 1)
    def _epilogue():
        ht_ref[...] = jax.nn.sigmoid(acc_ref[...] + mut_ref[...])


def _mm_call(vt, rt, mut):
    return pl.pallas_call(
        _mm_body,
        grid=(_NKB,),
        in_specs=[
            pl.BlockSpec((D, _KB), lambda k: (0, k)),
            pl.BlockSpec((_KB, M), lambda k: (k, 0)),
            pl.BlockSpec((D, 1), lambda k: (0, 0)),
        ],
        out_specs=pl.BlockSpec((D, M), lambda k: (0, 0)),
        out_shape=jax.ShapeDtypeStruct((D, M), jnp.float32),
        scratch_shapes=[pltpu.VMEM((D, M), jnp.float32)],
        compiler_params=pltpu.CompilerParams(
            dimension_semantics=("arbitrary",),
        ),
    )(vt, rt, mut)


def _combine_body(ht_ref, a2t_ref, bj_ref, out_ref):
    at = a2t_ref[0] + a2t_ref[1]
    s = jnp.sum(ht_ref[...] * at)
    out_ref[...] = s + bj_ref[...]


def _combine_call(ht, a2t, bj):
    return pl.pallas_call(
        _combine_body,
        in_specs=[
            pl.BlockSpec((D, M), lambda: (0, 0)),
            pl.BlockSpec((_NC, D, M), lambda: (0, 0, 0)),
            pl.BlockSpec((B,), lambda: (0,)),
        ],
        out_specs=pl.BlockSpec((B,), lambda: (0,)),
        out_shape=jax.ShapeDtypeStruct((B,), jnp.float32),
    )(ht, a2t, bj)


def kernel(r, i, j, v, mu, w, b):
    i1 = i.astype(jnp.int32)
    j1 = j.astype(jnp.int32)
    wf = w.T.reshape(D * N)
    zeros = jnp.zeros((D * M,), jnp.float32)
    a2, bj = _sc_call()(i1, j1, wf, b, zeros)
    ht = _mm_call(v.T, r.T, mu.T)
    return _combine_call(ht, a2.reshape(_NC, D, M), bj)
